# baseline (speedup 1.0000x reference)
import functools

import jax
import jax.numpy as jnp
from jax import lax
from jax.experimental import pallas as pl
from jax.experimental.pallas import tpu as pltpu
from jax.experimental.pallas import tpu_sc as plsc

_N = 10000
_E = 320000
_D = 128

_NC = 2
_NS = 16
_NW = _NC * _NS

_NPAD = 10240
_RPT = _NPAD // _NS
_CHUNK = 128
_NCH = 80
_EPT = _NCH * _CHUNK
_EPAD = _EPT * _NW
_PAD_IDX = 10048

_mesh = plsc.VectorSubcoreMesh(core_axis_name="c", subcore_axis_name="s")


@functools.partial(
    pl.kernel,
    out_type=jax.ShapeDtypeStruct((_NW, _NPAD), jnp.float32),
    mesh=_mesh,
    compiler_params=pltpu.CompilerParams(needs_layout_passes=False),
    scratch_types=[
        pltpu.VMEM((_NPAD,), jnp.float32),
        pltpu.VMEM((_CHUNK,), jnp.int32),
    ],
)
def _deg_kernel(dst_hbm, parts_hbm, hist, didx):
    c = lax.axis_index("c")
    s = lax.axis_index("s")
    wid = c * _NS + s
    zeros = jnp.zeros((16,), jnp.float32)

    def zero_body(i, carry):
        hist[pl.ds(i * 16, 16)] = zeros
        return carry

    lax.fori_loop(0, _NPAD // 16, zero_body, 0)

    ones = jnp.ones((16,), jnp.float32)
    base = wid * _EPT

    def edge_body(k, carry):
        pltpu.sync_copy(dst_hbm.at[pl.ds(base + k * _CHUNK, _CHUNK)], didx)
        for grp in range(_CHUNK // 16):
            idx = didx[pl.ds(grp * 16, 16)]
            plsc.addupdate_scatter(hist, [idx], ones)
        return carry

    lax.fori_loop(0, _EPT // _CHUNK, edge_body, 0)
    pltpu.sync_copy(hist, parts_hbm.at[wid])


@functools.partial(
    pl.kernel,
    out_type=jax.ShapeDtypeStruct((_NC * _NPAD, _D), jnp.float32),
    mesh=_mesh,
    scratch_types=[
        pltpu.VMEM_SHARED((_NPAD, _D), jnp.float32),
        pltpu.VMEM((_CHUNK, _D), jnp.float32),
        pltpu.VMEM((_CHUNK, _D), jnp.float32),
        pltpu.VMEM((_CHUNK,), jnp.int32),
        pltpu.VMEM((_CHUNK,), jnp.int32),
        pltpu.VMEM((_CHUNK,), jnp.int32),
        pltpu.VMEM((_CHUNK,), jnp.int32),
        pltpu.SemaphoreType.DMA,
        pltpu.SemaphoreType.DMA,
        pltpu.SemaphoreType.DMA,
        pltpu.SemaphoreType.DMA,
    ],
)
def _mp_kernel(g_hbm, src_hbm, dst_hbm, acc_hbm, acc_sh, rows0, rows1,
               sidx0, didx0, sidx1, didx1, sem_a, sem_b, sem_ia, sem_ib):
    c = lax.axis_index("c")
    s = lax.axis_index("s")
    wid = c * _NS + s
    zeros = jnp.zeros((16,), jnp.float32)

    def zero_body(r, carry):
        for col in range(_D // 16):
            rows0[r, pl.ds(col * 16, 16)] = zeros
        return carry

    lax.fori_loop(0, _CHUNK, zero_body, 0)

    row0 = s * _RPT
    for k in range(_RPT // _CHUNK):
        pltpu.sync_copy(rows0, acc_sh.at[pl.ds(row0 + k * _CHUNK, _CHUNK)])
    plsc.subcore_barrier()

    base = wid * _EPT

    def idx_load(k, si, di, sem):
        off = base + k * _CHUNK
        pltpu.async_copy(src_hbm.at[pl.ds(off, _CHUNK)], si, sem)
        pltpu.async_copy(dst_hbm.at[pl.ds(off, _CHUNK)], di, sem)

    def idx_wait(si, di, sem):
        pltpu.make_async_copy(src_hbm.at[pl.ds(base, _CHUNK)], si, sem).wait()
        pltpu.make_async_copy(dst_hbm.at[pl.ds(base, _CHUNK)], di, sem).wait()

    pltpu.sync_copy(src_hbm.at[pl.ds(base, _CHUNK)], sidx0)
    pltpu.sync_copy(dst_hbm.at[pl.ds(base, _CHUNK)], didx0)
    pltpu.async_copy(g_hbm.at[sidx0], rows0, sem_a)

    def edge_body(j, carry):
        k = 2 * j
        idx_load(k + 1, sidx1, didx1, sem_ib)
        pltpu.make_async_copy(g_hbm.at[sidx0], rows0, sem_a).wait()
        idx_wait(sidx1, didx1, sem_ib)
        pltpu.async_copy(g_hbm.at[sidx1], rows1, sem_b)
        pltpu.sync_copy(rows0, acc_sh.at[didx0], add=True)
        idx_load((k + 2) % _NCH, sidx0, didx0, sem_ia)
        pltpu.make_async_copy(g_hbm.at[sidx1], rows1, sem_b).wait()
        idx_wait(sidx0, didx0, sem_ia)
        pltpu.async_copy(g_hbm.at[sidx0], rows0, sem_a)
        pltpu.sync_copy(rows1, acc_sh.at[didx1], add=True)
        return carry

    lax.fori_loop(0, _NCH // 2, edge_body, 0)
    pltpu.make_async_copy(g_hbm.at[sidx0], rows0, sem_a).wait()
    plsc.subcore_barrier()

    for k in range(_RPT // _CHUNK):
        r0 = row0 + k * _CHUNK
        pltpu.sync_copy(acc_sh.at[pl.ds(r0, _CHUNK)], rows0)
        pltpu.sync_copy(rows0, acc_hbm.at[pl.ds(c * _NPAD + r0, _CHUNK)])


_RB = 80
_NG = _N // _RB
_ABLK = _NPAD // _RB


def _ms_body(parts_ref, x_ref, w_ref, g_ref, d_ref):
    deg = jnp.sum(parts_ref[...], axis=1, keepdims=True) + 1.0
    dinv = lax.rsqrt(deg)
    h = jnp.dot(x_ref[...], w_ref[...], preferred_element_type=jnp.float32)
    g_ref[...] = h * dinv
    d_ref[...] = dinv


def _ms_call(parts_t, x, w):
    return pl.pallas_call(
        _ms_body,
        grid=(_NG,),
        in_specs=[
            pl.BlockSpec((_RB, _NW), lambda i: (i, 0)),
            pl.BlockSpec((_RB, _D), lambda i: (i, 0)),
            pl.BlockSpec((_D, _D), lambda i: (0, 0)),
        ],
        out_specs=[
            pl.BlockSpec((_RB, _D), lambda i: (i, 0)),
            pl.BlockSpec((_RB, 1), lambda i: (i, 0)),
        ],
        out_shape=[
            jax.ShapeDtypeStruct((_N, _D), jnp.float32),
            jax.ShapeDtypeStruct((_N, 1), jnp.float32),
        ],
    )(parts_t, x, w)


def _cb_body(a0_ref, a1_ref, g_ref, d_ref, b_ref, w_ref, o_ref):
    acc = a0_ref[...] + a1_ref[...] + g_ref[...]
    p = jnp.maximum(acc * d_ref[...] + b_ref[...], 0.0)
    o_ref[...] = jnp.dot(p, w_ref[...], preferred_element_type=jnp.float32) * d_ref[...]


def _cb_call(acc, g, dinv_col, bias_row, w):
    return pl.pallas_call(
        _cb_body,
        grid=(_NG,),
        in_specs=[
            pl.BlockSpec((_RB, _D), lambda i: (i, 0)),
            pl.BlockSpec((_RB, _D), lambda i: (i + _ABLK, 0)),
            pl.BlockSpec((_RB, _D), lambda i: (i, 0)),
            pl.BlockSpec((_RB, 1), lambda i: (i, 0)),
            pl.BlockSpec((1, _D), lambda i: (0, 0)),
            pl.BlockSpec((_D, _D), lambda i: (0, 0)),
        ],
        out_specs=pl.BlockSpec((_RB, _D), lambda i: (i, 0)),
        out_shape=jax.ShapeDtypeStruct((_N, _D), jnp.float32),
    )(acc, acc, g, dinv_col, bias_row, w)


def _fin_body(a0_ref, a1_ref, g_ref, d_ref, b_ref, o_ref):
    acc = a0_ref[...] + a1_ref[...] + g_ref[...]
    o_ref[...] = acc * d_ref[...] + b_ref[...]


def _fin_call(acc, g, dinv_col, bias_row):
    return pl.pallas_call(
        _fin_body,
        grid=(_NG,),
        in_specs=[
            pl.BlockSpec((_RB, _D), lambda i: (i, 0)),
            pl.BlockSpec((_RB, _D), lambda i: (i + _ABLK, 0)),
            pl.BlockSpec((_RB, _D), lambda i: (i, 0)),
            pl.BlockSpec((_RB, 1), lambda i: (i, 0)),
            pl.BlockSpec((1, _D), lambda i: (0, 0)),
        ],
        out_specs=pl.BlockSpec((_RB, _D), lambda i: (i, 0)),
        out_shape=jax.ShapeDtypeStruct((_N, _D), jnp.float32),
    )(acc, acc, g, dinv_col, bias_row)


def kernel(x, edge_index, W1, b1, W2, b2, W3, b3):
    npad = _EPAD - _E
    pad_src = jnp.arange(npad, dtype=jnp.int32) % _N
    pad_dst = _N + jnp.arange(npad, dtype=jnp.int32) % (_NPAD - _N)
    src = jnp.concatenate([edge_index[0].astype(jnp.int32), pad_src])
    dst = jnp.concatenate([edge_index[1].astype(jnp.int32), pad_dst])

    parts_t = _deg_kernel(dst).T

    g1, dinv_col = _ms_call(parts_t, x, W1)
    a1 = _mp_kernel(g1, src, dst)
    g2 = _cb_call(a1, g1, dinv_col, b1.reshape(1, _D), W2)
    a2 = _mp_kernel(g2, src, dst)
    g3 = _cb_call(a2, g2, dinv_col, b2.reshape(1, _D), W3)
    a3 = _mp_kernel(g3, src, dst)
    return _fin_call(a3, g3, dinv_col, b3.reshape(1, _D))


# speedup vs baseline: 1.4659x; 1.4659x over previous
import functools

import jax
import jax.numpy as jnp
from jax import lax
from jax.experimental import pallas as pl
from jax.experimental.pallas import tpu as pltpu
from jax.experimental.pallas import tpu_sc as plsc

_N = 10000
_E = 320000
_D = 128

_NC = 2
_NS = 16
_NW = _NC * _NS

_NPAD = 10240
_RPT = _NPAD // _NS
_CHUNK = 128
_NCH = 80
_EPT = _NCH * _CHUNK
_EPAD = _EPT * _NW
_PAD_IDX = 10048

_mesh = plsc.VectorSubcoreMesh(core_axis_name="c", subcore_axis_name="s")


@functools.partial(
    pl.kernel,
    out_type=jax.ShapeDtypeStruct((_NW, _NPAD), jnp.float32),
    mesh=_mesh,
    compiler_params=pltpu.CompilerParams(needs_layout_passes=False),
    scratch_types=[
        pltpu.VMEM((_NPAD,), jnp.float32),
        pltpu.VMEM((_CHUNK,), jnp.int32),
    ],
)
def _deg_kernel(dst_hbm, parts_hbm, hist, didx):
    c = lax.axis_index("c")
    s = lax.axis_index("s")
    wid = c * _NS + s
    zeros = jnp.zeros((16,), jnp.float32)

    def zero_body(i, carry):
        hist[pl.ds(i * 16, 16)] = zeros
        return carry

    lax.fori_loop(0, _NPAD // 16, zero_body, 0)

    ones = jnp.ones((16,), jnp.float32)
    base = wid * _EPT

    def edge_body(k, carry):
        pltpu.sync_copy(dst_hbm.at[pl.ds(base + k * _CHUNK, _CHUNK)], didx)
        for grp in range(_CHUNK // 16):
            idx = didx[pl.ds(grp * 16, 16)]
            plsc.addupdate_scatter(hist, [idx], ones)
        return carry

    lax.fori_loop(0, _EPT // _CHUNK, edge_body, 0)
    pltpu.sync_copy(hist, parts_hbm.at[wid])


@functools.partial(
    pl.kernel,
    out_type=jax.ShapeDtypeStruct((_NC * _NPAD, _D), jnp.float32),
    mesh=_mesh,
    scratch_types=[
        pltpu.VMEM_SHARED((_NPAD, _D), jnp.float32),
        pltpu.VMEM((_CHUNK, _D), jnp.float32),
        pltpu.VMEM((_CHUNK, _D), jnp.float32),
        pltpu.VMEM((_CHUNK,), jnp.int32),
        pltpu.VMEM((_CHUNK,), jnp.int32),
        pltpu.VMEM((_CHUNK,), jnp.int32),
        pltpu.VMEM((_CHUNK,), jnp.int32),
        pltpu.SemaphoreType.DMA,
        pltpu.SemaphoreType.DMA,
        pltpu.SemaphoreType.DMA,
        pltpu.SemaphoreType.DMA,
    ],
)
def _mp_kernel(g_hbm, src_hbm, dst_hbm, acc_hbm, acc_sh, rows0, rows1,
               sidx0, didx0, sidx1, didx1, sem_a, sem_b, sem_ia, sem_ib):
    c = lax.axis_index("c")
    s = lax.axis_index("s")
    wid = c * _NS + s
    zeros = jnp.zeros((16,), jnp.float32)

    def zero_body(r, carry):
        for col in range(_D // 16):
            rows0[r, pl.ds(col * 16, 16)] = zeros
        return carry

    lax.fori_loop(0, _CHUNK, zero_body, 0)

    row0 = s * _RPT
    for k in range(_RPT // _CHUNK):
        pltpu.sync_copy(rows0, acc_sh.at[pl.ds(row0 + k * _CHUNK, _CHUNK)])
    plsc.subcore_barrier()

    base = wid * _EPT

    def idx_load(k, si, di, sem):
        off = base + k * _CHUNK
        pltpu.async_copy(src_hbm.at[pl.ds(off, _CHUNK)], si, sem)
        pltpu.async_copy(dst_hbm.at[pl.ds(off, _CHUNK)], di, sem)

    def idx_wait(si, di, sem):
        pltpu.make_async_copy(src_hbm.at[pl.ds(base, _CHUNK)], si, sem).wait()
        pltpu.make_async_copy(dst_hbm.at[pl.ds(base, _CHUNK)], di, sem).wait()

    pltpu.sync_copy(src_hbm.at[pl.ds(base, _CHUNK)], sidx0)
    pltpu.sync_copy(dst_hbm.at[pl.ds(base, _CHUNK)], didx0)
    pltpu.async_copy(g_hbm.at[sidx0], rows0, sem_a)

    def edge_body(j, carry):
        k = 2 * j
        idx_load(k + 1, sidx1, didx1, sem_ib)
        pltpu.make_async_copy(g_hbm.at[sidx0], rows0, sem_a).wait()
        idx_wait(sidx1, didx1, sem_ib)
        pltpu.async_copy(g_hbm.at[sidx1], rows1, sem_b)
        pltpu.sync_copy(rows0, acc_sh.at[didx0], add=True)
        idx_load((k + 2) % _NCH, sidx0, didx0, sem_ia)
        pltpu.make_async_copy(g_hbm.at[sidx1], rows1, sem_b).wait()
        idx_wait(sidx0, didx0, sem_ia)
        pltpu.async_copy(g_hbm.at[sidx0], rows0, sem_a)
        pltpu.sync_copy(rows1, acc_sh.at[didx1], add=True)
        return carry

    lax.fori_loop(0, _NCH // 2, edge_body, 0)
    pltpu.make_async_copy(g_hbm.at[sidx0], rows0, sem_a).wait()
    plsc.subcore_barrier()

    for k in range(_RPT // _CHUNK):
        r0 = row0 + k * _CHUNK
        pltpu.sync_copy(acc_sh.at[pl.ds(r0, _CHUNK)], rows0)
        pltpu.sync_copy(rows0, acc_hbm.at[pl.ds(c * _NPAD + r0, _CHUNK)])


_RB = 1000
_NG = _N // _RB


def _mm_body(x_ref, w_ref, h_ref):
    h_ref[...] = jnp.dot(x_ref[...], w_ref[...],
                         preferred_element_type=jnp.float32)


def _mm_call(x, w):
    return pl.pallas_call(
        _mm_body,
        grid=(_NG,),
        in_specs=[
            pl.BlockSpec((_RB, _D), lambda i: (i, 0)),
            pl.BlockSpec((_D, _D), lambda i: (0, 0)),
        ],
        out_specs=pl.BlockSpec((_RB, _D), lambda i: (i, 0)),
        out_shape=jax.ShapeDtypeStruct((_N, _D), jnp.float32),
    )(x, w)


def _sc_body(parts_ref, h_ref, g_ref, d_ref):
    deg = jnp.sum(parts_ref[...], axis=1, keepdims=True) + 1.0
    dinv = lax.rsqrt(deg)
    g_ref[...] = h_ref[...] * dinv
    d_ref[...] = dinv


def _sc_call(parts_t, h):
    return pl.pallas_call(
        _sc_body,
        grid=(_NG,),
        in_specs=[
            pl.BlockSpec((_RB, _NW), lambda i: (i, 0)),
            pl.BlockSpec((_RB, _D), lambda i: (i, 0)),
        ],
        out_specs=[
            pl.BlockSpec((_RB, _D), lambda i: (i, 0)),
            pl.BlockSpec((_RB, 1), lambda i: (i, 0)),
        ],
        out_shape=[
            jax.ShapeDtypeStruct((_N, _D), jnp.float32),
            jax.ShapeDtypeStruct((_N, 1), jnp.float32),
        ],
    )(parts_t, h)


def _cb_body(a0_ref, a1_ref, g_ref, d_ref, b_ref, w_ref, o_ref):
    acc = a0_ref[0] + a1_ref[0] + g_ref[...]
    p = jnp.maximum(acc * d_ref[...] + b_ref[...], 0.0)
    o_ref[...] = jnp.dot(p, w_ref[...], preferred_element_type=jnp.float32) * d_ref[...]


def _cb_call(acc3, g, dinv_col, bias_row, w):
    return pl.pallas_call(
        _cb_body,
        grid=(_NG,),
        in_specs=[
            pl.BlockSpec((1, _RB, _D), lambda i: (0, i, 0)),
            pl.BlockSpec((1, _RB, _D), lambda i: (1, i, 0)),
            pl.BlockSpec((_RB, _D), lambda i: (i, 0)),
            pl.BlockSpec((_RB, 1), lambda i: (i, 0)),
            pl.BlockSpec((1, _D), lambda i: (0, 0)),
            pl.BlockSpec((_D, _D), lambda i: (0, 0)),
        ],
        out_specs=pl.BlockSpec((_RB, _D), lambda i: (i, 0)),
        out_shape=jax.ShapeDtypeStruct((_N, _D), jnp.float32),
    )(acc3, acc3, g, dinv_col, bias_row, w)


def _fin_body(a0_ref, a1_ref, g_ref, d_ref, b_ref, o_ref):
    acc = a0_ref[0] + a1_ref[0] + g_ref[...]
    o_ref[...] = acc * d_ref[...] + b_ref[...]


def _fin_call(acc3, g, dinv_col, bias_row):
    return pl.pallas_call(
        _fin_body,
        grid=(_NG,),
        in_specs=[
            pl.BlockSpec((1, _RB, _D), lambda i: (0, i, 0)),
            pl.BlockSpec((1, _RB, _D), lambda i: (1, i, 0)),
            pl.BlockSpec((_RB, _D), lambda i: (i, 0)),
            pl.BlockSpec((_RB, 1), lambda i: (i, 0)),
            pl.BlockSpec((1, _D), lambda i: (0, 0)),
        ],
        out_specs=pl.BlockSpec((_RB, _D), lambda i: (i, 0)),
        out_shape=jax.ShapeDtypeStruct((_N, _D), jnp.float32),
    )(acc3, acc3, g, dinv_col, bias_row)


def kernel(x, edge_index, W1, b1, W2, b2, W3, b3):
    npad = _EPAD - _E
    pad_src = jnp.arange(npad, dtype=jnp.int32) % _N
    pad_dst = _N + jnp.arange(npad, dtype=jnp.int32) % (_NPAD - _N)
    src = jnp.concatenate([edge_index[0].astype(jnp.int32), pad_src])
    dst = jnp.concatenate([edge_index[1].astype(jnp.int32), pad_dst])

    h1 = _mm_call(x, W1)
    parts_t = _deg_kernel(dst).T

    g1, dinv_col = _sc_call(parts_t, h1)
    a1 = _mp_kernel(g1, src, dst).reshape(_NC, _NPAD, _D)
    g2 = _cb_call(a1, g1, dinv_col, b1.reshape(1, _D), W2)
    a2 = _mp_kernel(g2, src, dst).reshape(_NC, _NPAD, _D)
    g3 = _cb_call(a2, g2, dinv_col, b2.reshape(1, _D), W3)
    a3 = _mp_kernel(g3, src, dst).reshape(_NC, _NPAD, _D)
    return _fin_call(a3, g3, dinv_col, b3.reshape(1, _D))
